# Initial kernel scaffold; baseline (speedup 1.0000x reference)
import jax
import jax.numpy as jnp
from jax.experimental import pallas as pl


def kernel(x, edge_index, initial_weight, W_ih, W_hh, b_ih, b_hh):
    raise NotImplementedError("write your pallas kernel here")


# speedup vs baseline: 32.8048x; 32.8048x over previous
import functools

import jax
import jax.numpy as jnp
from jax import lax
from jax.experimental import pallas as pl
from jax.experimental.pallas import tpu as pltpu
from jax.experimental.pallas import tpu_sc as plsc

N = 50000
E = 800000
IN_DIM = 64
OUT_DIM = 64
FLAT = IN_DIM * OUT_DIM
FLAT3 = 3 * FLAT

NC = 2
NS = 16
L = 16

HALF = N // NC
DUMMY = HALF
ACC_ROWS = HALF + 8

EDGES_PER_TILE = E // (NC * NS)
A_CH = 2000
A_NFULL = 12
A_TAIL = EDGES_PER_TILE - A_NFULL * A_CH
A_PAD = 25088
DEG_SH = 50176
DEG_SLICE = DEG_SH // NS

EDGES_PER_SUB = E // NS
C_CH = 2000
C_NCH = EDGES_PER_SUB // C_CH
C_BUF = 2304
C_TRASH = 2288
G = 128


def _deg_body(row_ref, col_ref, degp_ref, deg_sh, zbuf, rstage, cstage, valbuf,
              colbuf2, sem_s):
  c = lax.axis_index("c")
  s = lax.axis_index("s")
  tile = c * NS + s
  base_e = tile * EDGES_PER_TILE

  zeros16 = jnp.zeros((L,), jnp.float32)
  for g in range(DEG_SLICE // L):
    zbuf[pl.ds(g * L, L)] = zeros16
  pltpu.sync_copy(zbuf, deg_sh.at[pl.ds(s * DEG_SLICE, DEG_SLICE)])
  plsc.subcore_barrier()

  lane = lax.iota(jnp.int32, L)

  def scan_chunk(off_e, off_buf, n_groups):
    pltpu.sync_copy(row_ref.at[pl.ds(off_e, A_CH)],
                    rstage.at[pl.ds(0, A_CH)])
    pltpu.sync_copy(col_ref.at[pl.ds(off_e, A_CH)],
                    cstage.at[pl.ds(0, A_CH)])

    def grp(j, _):
      r = rstage[pl.ds(j * L, L)]
      cv = cstage[pl.ds(j * L, L)]
      val = jnp.where(r != cv, 1.0, 0.0).astype(jnp.float32)
      off = off_buf + j * L
      valbuf[pl.ds(off, L)] = val
      colbuf2[lax.shift_right_logical(off, 7), pl.ds(lax.bitwise_and(off, G - 1), L)] = cv
      return 0

    lax.fori_loop(0, n_groups, grp, 0)

  def chunk_loop(k, _):
    scan_chunk(base_e + k * A_CH, k * A_CH, A_CH // L)
    return 0

  lax.fori_loop(0, A_NFULL, chunk_loop, 0)

  toff = base_e + A_NFULL * A_CH
  tbuf = A_NFULL * A_CH
  pltpu.sync_copy(row_ref.at[pl.ds(toff, A_TAIL)],
                  rstage.at[pl.ds(0, A_TAIL)])
  pltpu.sync_copy(col_ref.at[pl.ds(toff, A_TAIL)],
                  cstage.at[pl.ds(0, A_TAIL)])

  def tgrp(j, _):
    r = rstage[pl.ds(j * L, L)]
    cv = cstage[pl.ds(j * L, L)]
    val = jnp.where(r != cv, 1.0, 0.0).astype(jnp.float32)
    off = tbuf + j * L
    valbuf[pl.ds(off, L)] = val
    colbuf2[off // G, pl.ds(off % G, L)] = cv
    return 0

  lax.fori_loop(0, (A_TAIL - 8) // L, tgrp, 0)

  jt = (A_TAIL - 8) // L
  r = rstage[pl.ds(jt * L, L)]
  cv = cstage[pl.ds(jt * L, L)]
  ok = lane < 8
  val = jnp.where((r != cv) & ok, 1.0, 0.0).astype(jnp.float32)
  cv = jnp.where(ok, cv, 0)
  offt = tbuf + jt * L
  valbuf[pl.ds(offt, L)] = val
  colbuf2[lax.shift_right_logical(offt, 7), pl.ds(lax.bitwise_and(offt, G - 1), L)] = cv

  zi16 = jnp.zeros((L,), jnp.int32)
  for t in range((A_PAD - (EDGES_PER_TILE + 8)) // L):
    flat = EDGES_PER_TILE + 8 + t * L
    valbuf[pl.ds(flat, L)] = zeros16
    colbuf2[flat // G, pl.ds(flat % G, L)] = zi16

  def scat(g, _):
    pltpu.async_copy(valbuf.at[pl.ds(g * G, G)], deg_sh.at[colbuf2.at[g]],
                     sem_s, add=True)
    return 0

  lax.fori_loop(0, A_PAD // G, scat, 0)

  def drain(g, _):
    pltpu.make_async_copy(valbuf.at[pl.ds(0, G)], deg_sh.at[colbuf2.at[0]],
                          sem_s).wait()
    return 0

  lax.fori_loop(0, A_PAD // G, drain, 0)
  plsc.subcore_barrier()

  pltpu.sync_copy(deg_sh.at[pl.ds(s * DEG_SLICE, DEG_SLICE)], zbuf)

  @pl.when(s < NS - 1)
  def _():
    pltpu.sync_copy(zbuf, degp_ref.at[pl.ds(c * N + s * DEG_SLICE, DEG_SLICE)])

  @pl.when(s == NS - 1)
  def _():
    last = N - (NS - 1) * DEG_SLICE
    pltpu.sync_copy(zbuf.at[pl.ds(0, last)],
                    degp_ref.at[pl.ds(c * N + (NS - 1) * DEG_SLICE, last)])


def _deg_call(row, col):
  kfn = pl.kernel(
      _deg_body,
      out_type=jax.ShapeDtypeStruct((NC * N,), jnp.float32),
      mesh=plsc.VectorSubcoreMesh(core_axis_name="c", subcore_axis_name="s"),
      scratch_types=[
          pltpu.VMEM_SHARED((DEG_SH,), jnp.float32),
          pltpu.VMEM((DEG_SLICE,), jnp.float32),
          pltpu.VMEM((A_CH + 16,), jnp.int32),
          pltpu.VMEM((A_CH + 16,), jnp.int32),
          pltpu.VMEM((A_PAD,), jnp.float32),
          pltpu.VMEM((A_PAD // G, G), jnp.int32),
          pltpu.SemaphoreType.DMA,
      ],
  )
  return kfn(row, col)


def _newton_rsqrt(x):
  i = plsc.bitcast(x, jnp.int32)
  y = plsc.bitcast(jnp.int32(0x5F3759DF) - lax.shift_right_logical(i, 1),
                   jnp.float32)
  y = y * (1.5 - 0.5 * x * y * y)
  y = y * (1.5 - 0.5 * x * y * y)
  y = y * (1.5 - 0.5 * x * y * y)
  return y


def _scatter_body(row_ref, col_ref, hp_ref, degp_ref, out_ref, accum, rstage,
                  cstage, rbuf, cbuf2, grows, grows_b, dbuf, dstage0, dstage1,
                  sem_r, sem_c, sem_ga, sem_gb):
  c = lax.axis_index("c")
  s = lax.axis_index("s")
  base = c * HALF

  def init_chunk(r0, nrows):
    pltpu.sync_copy(hp_ref.at[pl.ds(base + r0, nrows)],
                    grows.at[pl.ds(0, nrows)])
    pltpu.sync_copy(grows.at[pl.ds(0, nrows)], accum.at[pl.ds(r0, nrows)])

  @pl.when(s < NS - 1)
  def _():
    def body(q, _):
      init_chunk(s * 1568 + q * G, G)
      return 0
    lax.fori_loop(0, 12, body, 0)
    init_chunk(s * 1568 + 12 * G, 32)

  @pl.when(s == NS - 1)
  def _():
    def body(q, _):
      init_chunk(15 * 1568 + q * G, G)
      return 0
    lax.fori_loop(0, 11, body, 0)
    init_chunk(15 * 1568 + 11 * G, 72)

  plsc.subcore_barrier()

  zeros16 = jnp.zeros((L,), jnp.int32)
  dummy16 = jnp.full((L,), DUMMY, jnp.int32)
  lane = lax.iota(jnp.int32, L)

  def wait_gather(buf, sem):
    pltpu.make_async_copy(hp_ref.at[rbuf.at[pl.ds(0, G)]], buf, sem).wait()

  def chunk(k, p_in):
    off = s * EDGES_PER_SUB + k * C_CH
    d1 = pltpu.async_copy(row_ref.at[pl.ds(off, C_CH)], rstage, sem_r)
    d2 = pltpu.async_copy(col_ref.at[pl.ds(off, C_CH)], cstage, sem_c)
    d1.wait()
    d2.wait()

    def grp(j, p):
      r = rstage[pl.ds(j * L, L)]
      cv = cstage[pl.ds(j * L, L)]
      cl = cv - base
      m = (cl >= 0) & (cl < HALF) & (r != cv)
      cum = plsc.cumsum(m.astype(jnp.int32))
      pos = jnp.where(m, p + cum - 1, C_TRASH + lane)
      plsc.store_scatter(rbuf, [pos], r)
      plsc.store_scatter(cbuf2, [lax.shift_right_logical(pos, 7),
                                  lax.bitwise_and(pos, G - 1)], cl)
      return p + jnp.max(cum)

    cnt = lax.fori_loop(0, C_CH // L, grp, p_in)
    ngf = cnt // G

    @pl.when(ngf > 0)
    def _():
      pltpu.async_copy(hp_ref.at[rbuf.at[pl.ds(0, G)]], grows, sem_ga)

      def do_group(g, _):
        even = (g % 2) == 0

        @pl.when(g + 1 < ngf)
        def _():
          @pl.when(even)
          def _():
            pltpu.async_copy(hp_ref.at[rbuf.at[pl.ds((g + 1) * G, G)]],
                             grows_b, sem_gb)

          @pl.when(jnp.logical_not(even))
          def _():
            pltpu.async_copy(hp_ref.at[rbuf.at[pl.ds((g + 1) * G, G)]],
                             grows, sem_ga)

        @pl.when(even)
        def _():
          wait_gather(grows, sem_ga)
          pltpu.sync_copy(grows, accum.at[cbuf2.at[g]], add=True)

        @pl.when(jnp.logical_not(even))
        def _():
          wait_gather(grows_b, sem_gb)
          pltpu.sync_copy(grows_b, accum.at[cbuf2.at[g]], add=True)

        return 0

      lax.fori_loop(0, ngf, do_group, 0)

      for t in range(G // L):
        tmp = rbuf[pl.ds(ngf * G + t * L, L)]
        rbuf[pl.ds(t * L, L)] = tmp
        tmp2 = cbuf2[ngf, pl.ds(t * L, L)]
        cbuf2[0, pl.ds(t * L, L)] = tmp2

    return cnt - ngf * G

  rem = lax.fori_loop(0, C_NCH, chunk, jnp.int32(0))


  @pl.when(s < NS - 1)
  def _():
    pltpu.sync_copy(degp_ref.at[pl.ds(base + s * 1568, 1568)], dstage0)
    pltpu.sync_copy(degp_ref.at[pl.ds(N + base + s * 1568, 1568)], dstage1)

  @pl.when(s == NS - 1)
  def _():
    pltpu.sync_copy(degp_ref.at[pl.ds(base + 15 * 1568, 1480)],
                    dstage0.at[pl.ds(0, 1480)])
    pltpu.sync_copy(degp_ref.at[pl.ds(N + base + 15 * 1568, 1480)],
                    dstage1.at[pl.ds(0, 1480)])

  @pl.when(rem > 0)
  def _():
    for t in range(G // L):
      padpos = rem + t * L + lane
      rbuf[pl.ds(rem + t * L, L)] = zeros16
      plsc.store_scatter(cbuf2, [lax.shift_right_logical(padpos, 7),
                                  lax.bitwise_and(padpos, G - 1)], dummy16)
    pltpu.sync_copy(hp_ref.at[rbuf.at[pl.ds(0, G)]], grows)
    pltpu.sync_copy(grows, accum.at[cbuf2.at[0]], add=True)

  plsc.subcore_barrier()

  def out_chunk(r0, lo, nrows):
    pltpu.sync_copy(accum.at[pl.ds(r0, nrows)], grows.at[pl.ds(0, nrows)])

    def dgrp(jj, _):
      d0 = dstage0[pl.ds(lo + jj * L, L)]
      d1 = dstage1[pl.ds(lo + jj * L, L)]
      dbuf[pl.ds(jj * L, L)] = _newton_rsqrt(d0 + d1 + 1.0)
      return 0

    lax.fori_loop(0, (nrows + L - 1) // L, dgrp, 0)

    def rowf(rr, _):
      g16 = lax.bitwise_and(rr, jnp.int32(~(L - 1)))
      dv = dbuf[pl.ds(g16, L)]
      dsp = jnp.max(jnp.where(lane == lax.bitwise_and(rr, jnp.int32(L - 1)),
                              dv, jnp.float32(-3e38)))
      for seg in range(OUT_DIM // L):
        v = grows[rr, pl.ds(seg * L, L)]
        grows[rr, pl.ds(seg * L, L)] = jnp.maximum(v * dsp, 0.0)
      return 0

    lax.fori_loop(0, nrows, rowf, 0)
    pltpu.sync_copy(grows.at[pl.ds(0, nrows)],
                    out_ref.at[pl.ds(base + r0, nrows)])

  @pl.when(s < NS - 1)
  def _():
    def body(q, _):
      out_chunk(s * 1568 + q * G, q * G, G)
      return 0
    lax.fori_loop(0, 12, body, 0)
    out_chunk(s * 1568 + 12 * G, 12 * G, 32)

  @pl.when(s == NS - 1)
  def _():
    def body(q, _):
      out_chunk(15 * 1568 + q * G, q * G, G)
      return 0
    lax.fori_loop(0, 11, body, 0)
    out_chunk(15 * 1568 + 11 * G, 11 * G, 72)


def _scatter_call(row, col, hp, degp_flat):
  kfn = pl.kernel(
      _scatter_body,
      out_type=jax.ShapeDtypeStruct((N, OUT_DIM), jnp.float32),
      mesh=plsc.VectorSubcoreMesh(core_axis_name="c", subcore_axis_name="s"),
      scratch_types=[
          pltpu.VMEM_SHARED((ACC_ROWS, OUT_DIM), jnp.float32),
          pltpu.VMEM((C_CH,), jnp.int32),
          pltpu.VMEM((C_CH,), jnp.int32),
          pltpu.VMEM((C_BUF,), jnp.int32),
          pltpu.VMEM((18, G), jnp.int32),
          pltpu.VMEM((G, OUT_DIM), jnp.float32),
          pltpu.VMEM((G, OUT_DIM), jnp.float32),
          pltpu.VMEM((G,), jnp.float32),
          pltpu.VMEM((1568,), jnp.float32),
          pltpu.VMEM((1568,), jnp.float32),
          pltpu.SemaphoreType.DMA,
          pltpu.SemaphoreType.DMA,
          pltpu.SemaphoreType.DMA,
          pltpu.SemaphoreType.DMA,
      ],
      compiler_params=pltpu.CompilerParams(needs_layout_passes=False,
                                           use_tc_tiling_on_sc=False),
  )
  return kfn(row, col, hp, degp_flat)


GRU_BLK = 512
GRU_NB = FLAT3 // GRU_BLK


def _gru_body(x_ref, wih_ref, whh_ref, bih_ref, bhh_ref, wout_ref, gi_s, gh_s):
  i = pl.program_id(0)
  xv = x_ref[...]
  dn = (((1,), (1,)), ((), ()))
  gi = lax.dot_general(xv, wih_ref[...], dn,
                       preferred_element_type=jnp.float32) + bih_ref[...]
  gh = lax.dot_general(xv, whh_ref[...], dn,
                       preferred_element_type=jnp.float32) + bhh_ref[...]
  gi_s[:, pl.ds(i * GRU_BLK, GRU_BLK)] = gi
  gh_s[:, pl.ds(i * GRU_BLK, GRU_BLK)] = gh

  @pl.when(i == GRU_NB - 1)
  def _():
    r = jax.nn.sigmoid(gi_s[:, 0:FLAT] + gh_s[:, 0:FLAT])
    z = jax.nn.sigmoid(gi_s[:, FLAT:2 * FLAT] + gh_s[:, FLAT:2 * FLAT])
    n = jnp.tanh(gi_s[:, 2 * FLAT:] + r * gh_s[:, 2 * FLAT:])
    wout_ref[...] = (1.0 - z) * n + z * xv


def _gru_call(x_flat, W_ih, W_hh, b_ih, b_hh):
  return pl.pallas_call(
      _gru_body,
      grid=(GRU_NB,),
      in_specs=[
          pl.BlockSpec((1, FLAT), lambda i: (0, 0)),
          pl.BlockSpec((GRU_BLK, FLAT), lambda i: (i, 0)),
          pl.BlockSpec((GRU_BLK, FLAT), lambda i: (i, 0)),
          pl.BlockSpec((1, GRU_BLK), lambda i: (0, i)),
          pl.BlockSpec((1, GRU_BLK), lambda i: (0, i)),
      ],
      out_specs=pl.BlockSpec((1, FLAT), lambda i: (0, 0)),
      out_shape=jax.ShapeDtypeStruct((1, FLAT), jnp.float32),
      scratch_shapes=[
          pltpu.VMEM((1, FLAT3), jnp.float32),
          pltpu.VMEM((1, FLAT3), jnp.float32),
      ],
  )(x_flat, W_ih, W_hh, b_ih, b_hh)


ROW_BLK = 2000
ROW_NB = N // ROW_BLK


def _hp_body(x_ref, w_ref, degp_ref, hp_ref):
  d = lax.rsqrt(degp_ref[:, 0] + degp_ref[:, 1] + 1.0)
  h = jnp.dot(x_ref[...], w_ref[...], preferred_element_type=jnp.float32)
  hp_ref[...] = h * d[:, None]


def _hp_call(x, weight, degp):
  return pl.pallas_call(
      _hp_body,
      grid=(ROW_NB,),
      in_specs=[
          pl.BlockSpec((ROW_BLK, IN_DIM), lambda i: (i, 0)),
          pl.BlockSpec((IN_DIM, OUT_DIM), lambda i: (0, 0)),
          pl.BlockSpec((ROW_BLK, NC), lambda i: (i, 0)),
      ],
      out_specs=pl.BlockSpec((ROW_BLK, OUT_DIM), lambda i: (i, 0)),
      out_shape=jax.ShapeDtypeStruct((N, OUT_DIM), jnp.float32),
  )(x, weight, degp)


def _final_body(acc_ref, degp_ref, out_ref):
  d = lax.rsqrt(degp_ref[:, 0] + degp_ref[:, 1] + 1.0)
  out_ref[...] = jnp.maximum(acc_ref[...] * d[:, None], 0.0)


def _final_call(accum, degp):
  return pl.pallas_call(
      _final_body,
      grid=(ROW_NB,),
      in_specs=[
          pl.BlockSpec((ROW_BLK, OUT_DIM), lambda i: (i, 0)),
          pl.BlockSpec((ROW_BLK, NC), lambda i: (i, 0)),
      ],
      out_specs=pl.BlockSpec((ROW_BLK, OUT_DIM), lambda i: (i, 0)),
      out_shape=jax.ShapeDtypeStruct((N, OUT_DIM), jnp.float32),
  )(accum, degp)


def kernel(x, edge_index, initial_weight, W_ih, W_hh, b_ih, b_hh):
  x_flat = initial_weight.reshape(1, FLAT)
  row, col = edge_index[0], edge_index[1]
  degp_flat = _deg_call(row, col)
  degp = degp_flat.reshape(NC, N).T
  wnew = _gru_call(x_flat, W_ih, W_hh, b_ih.reshape(1, FLAT3),
                   b_hh.reshape(1, FLAT3))
  hp = _hp_call(x, wnew.reshape(IN_DIM, OUT_DIM), degp)
  return _scatter_call(row, col, hp, degp_flat)


# speedup vs baseline: 34.1258x; 1.0403x over previous
import functools

import jax
import jax.numpy as jnp
from jax import lax
from jax.experimental import pallas as pl
from jax.experimental.pallas import tpu as pltpu
from jax.experimental.pallas import tpu_sc as plsc

N = 50000
E = 800000
IN_DIM = 64
OUT_DIM = 64
FLAT = IN_DIM * OUT_DIM
FLAT3 = 3 * FLAT

NC = 2
NS = 16
L = 16

HALF = N // NC
DUMMY = HALF
ACC_ROWS = HALF + 8

EDGES_PER_TILE = E // (NC * NS)
A_CH = 2000
A_NFULL = 12
A_TAIL = EDGES_PER_TILE - A_NFULL * A_CH
A_PAD = 25088
DEG_SH = 50176
DEG_SLICE = DEG_SH // NS

EDGES_PER_SUB = E // NS
C_CH = 2000
C_NCH = EDGES_PER_SUB // C_CH
C_BUF = 2304
C_TRASH = 2288
G = 128


def _deg_body(row_ref, col_ref, degp_ref, deg_sh, zbuf, rstage, cstage, valbuf,
              colbuf2, sem_s):
  c = lax.axis_index("c")
  s = lax.axis_index("s")
  tile = c * NS + s
  base_e = tile * EDGES_PER_TILE

  zeros16 = jnp.zeros((L,), jnp.float32)
  for g in range(DEG_SLICE // L):
    zbuf[pl.ds(g * L, L)] = zeros16
  pltpu.sync_copy(zbuf, deg_sh.at[pl.ds(s * DEG_SLICE, DEG_SLICE)])
  plsc.subcore_barrier()

  lane = lax.iota(jnp.int32, L)

  def scan_chunk(off_e, off_buf, n_groups):
    pltpu.sync_copy(row_ref.at[pl.ds(off_e, A_CH)],
                    rstage.at[pl.ds(0, A_CH)])
    pltpu.sync_copy(col_ref.at[pl.ds(off_e, A_CH)],
                    cstage.at[pl.ds(0, A_CH)])

    def grp(j, _):
      r = rstage[pl.ds(j * L, L)]
      cv = cstage[pl.ds(j * L, L)]
      val = jnp.where(r != cv, 1.0, 0.0).astype(jnp.float32)
      off = off_buf + j * L
      valbuf[pl.ds(off, L)] = val
      colbuf2[lax.shift_right_logical(off, 7), pl.ds(lax.bitwise_and(off, G - 1), L)] = cv
      return 0

    lax.fori_loop(0, n_groups, grp, 0)

  def chunk_loop(k, _):
    scan_chunk(base_e + k * A_CH, k * A_CH, A_CH // L)
    return 0

  lax.fori_loop(0, A_NFULL, chunk_loop, 0)

  toff = base_e + A_NFULL * A_CH
  tbuf = A_NFULL * A_CH
  pltpu.sync_copy(row_ref.at[pl.ds(toff, A_TAIL)],
                  rstage.at[pl.ds(0, A_TAIL)])
  pltpu.sync_copy(col_ref.at[pl.ds(toff, A_TAIL)],
                  cstage.at[pl.ds(0, A_TAIL)])

  def tgrp(j, _):
    r = rstage[pl.ds(j * L, L)]
    cv = cstage[pl.ds(j * L, L)]
    val = jnp.where(r != cv, 1.0, 0.0).astype(jnp.float32)
    off = tbuf + j * L
    valbuf[pl.ds(off, L)] = val
    colbuf2[off // G, pl.ds(off % G, L)] = cv
    return 0

  lax.fori_loop(0, (A_TAIL - 8) // L, tgrp, 0)

  jt = (A_TAIL - 8) // L
  r = rstage[pl.ds(jt * L, L)]
  cv = cstage[pl.ds(jt * L, L)]
  ok = lane < 8
  val = jnp.where((r != cv) & ok, 1.0, 0.0).astype(jnp.float32)
  cv = jnp.where(ok, cv, 0)
  offt = tbuf + jt * L
  valbuf[pl.ds(offt, L)] = val
  colbuf2[lax.shift_right_logical(offt, 7), pl.ds(lax.bitwise_and(offt, G - 1), L)] = cv

  zi16 = jnp.zeros((L,), jnp.int32)
  for t in range((A_PAD - (EDGES_PER_TILE + 8)) // L):
    flat = EDGES_PER_TILE + 8 + t * L
    valbuf[pl.ds(flat, L)] = zeros16
    colbuf2[flat // G, pl.ds(flat % G, L)] = zi16

  def scat(g, _):
    pltpu.async_copy(valbuf.at[pl.ds(g * G, G)], deg_sh.at[colbuf2.at[g]],
                     sem_s, add=True)
    return 0

  lax.fori_loop(0, A_PAD // G, scat, 0)

  def drain(g, _):
    pltpu.make_async_copy(valbuf.at[pl.ds(0, G)], deg_sh.at[colbuf2.at[0]],
                          sem_s).wait()
    return 0

  lax.fori_loop(0, A_PAD // G, drain, 0)
  plsc.subcore_barrier()

  pltpu.sync_copy(deg_sh.at[pl.ds(s * DEG_SLICE, DEG_SLICE)], zbuf)

  @pl.when(s < NS - 1)
  def _():
    pltpu.sync_copy(zbuf, degp_ref.at[pl.ds(c * N + s * DEG_SLICE, DEG_SLICE)])

  @pl.when(s == NS - 1)
  def _():
    last = N - (NS - 1) * DEG_SLICE
    pltpu.sync_copy(zbuf.at[pl.ds(0, last)],
                    degp_ref.at[pl.ds(c * N + (NS - 1) * DEG_SLICE, last)])


def _deg_call(row, col):
  kfn = pl.kernel(
      _deg_body,
      out_type=jax.ShapeDtypeStruct((NC * N,), jnp.float32),
      mesh=plsc.VectorSubcoreMesh(core_axis_name="c", subcore_axis_name="s"),
      scratch_types=[
          pltpu.VMEM_SHARED((DEG_SH,), jnp.float32),
          pltpu.VMEM((DEG_SLICE,), jnp.float32),
          pltpu.VMEM((A_CH + 16,), jnp.int32),
          pltpu.VMEM((A_CH + 16,), jnp.int32),
          pltpu.VMEM((A_PAD,), jnp.float32),
          pltpu.VMEM((A_PAD // G, G), jnp.int32),
          pltpu.SemaphoreType.DMA,
      ],
  )
  return kfn(row, col)


def _newton_rsqrt(x):
  i = plsc.bitcast(x, jnp.int32)
  y = plsc.bitcast(jnp.int32(0x5F3759DF) - lax.shift_right_logical(i, 1),
                   jnp.float32)
  y = y * (1.5 - 0.5 * x * y * y)
  y = y * (1.5 - 0.5 * x * y * y)
  y = y * (1.5 - 0.5 * x * y * y)
  return y


def _scatter_body(row_ref, col_ref, hp_ref, degp_ref, out_ref, accum, rstage,
                  cstage, rbuf, cbuf2, grows, grows_b, dbuf, dstage0, dstage1,
                  sem_r, sem_c, sem_ga, sem_gb):
  c = lax.axis_index("c")
  s = lax.axis_index("s")
  base = c * HALF

  def init_chunk(r0, nrows):
    pltpu.sync_copy(hp_ref.at[pl.ds(base + r0, nrows)],
                    grows.at[pl.ds(0, nrows)])
    pltpu.sync_copy(grows.at[pl.ds(0, nrows)], accum.at[pl.ds(r0, nrows)])

  @pl.when(s < NS - 1)
  def _():
    def body(q, _):
      init_chunk(s * 1568 + q * G, G)
      return 0
    lax.fori_loop(0, 12, body, 0)
    init_chunk(s * 1568 + 12 * G, 32)

  @pl.when(s == NS - 1)
  def _():
    def body(q, _):
      init_chunk(15 * 1568 + q * G, G)
      return 0
    lax.fori_loop(0, 11, body, 0)
    init_chunk(15 * 1568 + 11 * G, 72)

  plsc.subcore_barrier()

  zeros16 = jnp.zeros((L,), jnp.int32)
  dummy16 = jnp.full((L,), DUMMY, jnp.int32)
  lane = lax.iota(jnp.int32, L)

  def wait_gather(buf, sem):
    pltpu.make_async_copy(hp_ref.at[rbuf.at[pl.ds(0, G)]], buf, sem).wait()

  def chunk(k, p_in):
    off = s * EDGES_PER_SUB + k * C_CH
    d1 = pltpu.async_copy(row_ref.at[pl.ds(off, C_CH)], rstage, sem_r)
    d2 = pltpu.async_copy(col_ref.at[pl.ds(off, C_CH)], cstage, sem_c)
    d1.wait()
    d2.wait()

    def grp(j, p):
      r = rstage[pl.ds(j * L, L)]
      cv = cstage[pl.ds(j * L, L)]
      cl = cv - base
      m = (cl >= 0) & (cl < HALF) & (r != cv)
      cum = plsc.cumsum(m.astype(jnp.int32))
      pos = jnp.where(m, p + cum - 1, C_TRASH + lane)
      plsc.store_scatter(rbuf, [pos], r)
      plsc.store_scatter(cbuf2, [lax.shift_right_logical(pos, 7),
                                  lax.bitwise_and(pos, G - 1)], cl)
      return p + jnp.max(cum)

    cnt = lax.fori_loop(0, C_CH // L, grp, p_in)
    ngf = cnt // G

    @pl.when(ngf > 0)
    def _():
      pltpu.async_copy(hp_ref.at[rbuf.at[pl.ds(0, G)]], grows, sem_ga)

      def do_group(g, _):
        even = (g % 2) == 0

        @pl.when(g + 1 < ngf)
        def _():
          @pl.when(even)
          def _():
            pltpu.async_copy(hp_ref.at[rbuf.at[pl.ds((g + 1) * G, G)]],
                             grows_b, sem_gb)

          @pl.when(jnp.logical_not(even))
          def _():
            pltpu.async_copy(hp_ref.at[rbuf.at[pl.ds((g + 1) * G, G)]],
                             grows, sem_ga)

        @pl.when(even)
        def _():
          wait_gather(grows, sem_ga)
          pltpu.sync_copy(grows, accum.at[cbuf2.at[g]], add=True)

        @pl.when(jnp.logical_not(even))
        def _():
          wait_gather(grows_b, sem_gb)
          pltpu.sync_copy(grows_b, accum.at[cbuf2.at[g]], add=True)

        return 0

      lax.fori_loop(0, ngf, do_group, 0)

      for t in range(G // L):
        tmp = rbuf[pl.ds(ngf * G + t * L, L)]
        rbuf[pl.ds(t * L, L)] = tmp
        tmp2 = cbuf2[ngf, pl.ds(t * L, L)]
        cbuf2[0, pl.ds(t * L, L)] = tmp2

    return cnt - ngf * G

  rem = lax.fori_loop(0, C_NCH, chunk, jnp.int32(0))


  @pl.when(s < NS - 1)
  def _():
    pltpu.sync_copy(degp_ref.at[pl.ds(base + s * 1568, 1568)], dstage0)
    pltpu.sync_copy(degp_ref.at[pl.ds(N + base + s * 1568, 1568)], dstage1)

  @pl.when(s == NS - 1)
  def _():
    pltpu.sync_copy(degp_ref.at[pl.ds(base + 15 * 1568, 1480)],
                    dstage0.at[pl.ds(0, 1480)])
    pltpu.sync_copy(degp_ref.at[pl.ds(N + base + 15 * 1568, 1480)],
                    dstage1.at[pl.ds(0, 1480)])

  @pl.when(rem > 0)
  def _():
    for t in range(G // L):
      padpos = rem + t * L + lane
      rbuf[pl.ds(rem + t * L, L)] = zeros16
      plsc.store_scatter(cbuf2, [lax.shift_right_logical(padpos, 7),
                                  lax.bitwise_and(padpos, G - 1)], dummy16)
    pltpu.sync_copy(hp_ref.at[rbuf.at[pl.ds(0, G)]], grows)
    pltpu.sync_copy(grows, accum.at[cbuf2.at[0]], add=True)

  plsc.subcore_barrier()

  def out_chunk(r0, lo, nrows):
    pltpu.sync_copy(accum.at[pl.ds(r0, nrows)], grows.at[pl.ds(0, nrows)])

    def dgrp(jj, _):
      d0 = dstage0[pl.ds(lo + jj * L, L)]
      d1 = dstage1[pl.ds(lo + jj * L, L)]
      dbuf[pl.ds(jj * L, L)] = _newton_rsqrt(d0 + d1 + 1.0)
      return 0

    lax.fori_loop(0, (nrows + L - 1) // L, dgrp, 0)

    def rowf(rr, _):
      g16 = lax.bitwise_and(rr, jnp.int32(~(L - 1)))
      dv = dbuf[pl.ds(g16, L)]
      dsp = jnp.max(jnp.where(lane == lax.bitwise_and(rr, jnp.int32(L - 1)),
                              dv, jnp.float32(-3e38)))
      for seg in range(OUT_DIM // L):
        v = grows[rr, pl.ds(seg * L, L)]
        grows[rr, pl.ds(seg * L, L)] = jnp.maximum(v * dsp, 0.0)
      return 0

    lax.fori_loop(0, nrows, rowf, 0)
    pltpu.sync_copy(grows.at[pl.ds(0, nrows)],
                    out_ref.at[pl.ds(base + r0, nrows)])

  @pl.when(s < NS - 1)
  def _():
    def body(q, _):
      out_chunk(s * 1568 + q * G, q * G, G)
      return 0
    lax.fori_loop(0, 12, body, 0)
    out_chunk(s * 1568 + 12 * G, 12 * G, 32)

  @pl.when(s == NS - 1)
  def _():
    def body(q, _):
      out_chunk(15 * 1568 + q * G, q * G, G)
      return 0
    lax.fori_loop(0, 11, body, 0)
    out_chunk(15 * 1568 + 11 * G, 11 * G, 72)


def _scatter_call(row, col, hp, degp_flat):
  kfn = pl.kernel(
      _scatter_body,
      out_type=jax.ShapeDtypeStruct((N, OUT_DIM), jnp.float32),
      mesh=plsc.VectorSubcoreMesh(core_axis_name="c", subcore_axis_name="s"),
      scratch_types=[
          pltpu.VMEM_SHARED((ACC_ROWS, OUT_DIM), jnp.float32),
          pltpu.VMEM((C_CH,), jnp.int32),
          pltpu.VMEM((C_CH,), jnp.int32),
          pltpu.VMEM((C_BUF,), jnp.int32),
          pltpu.VMEM((18, G), jnp.int32),
          pltpu.VMEM((G, OUT_DIM), jnp.float32),
          pltpu.VMEM((G, OUT_DIM), jnp.float32),
          pltpu.VMEM((G,), jnp.float32),
          pltpu.VMEM((1568,), jnp.float32),
          pltpu.VMEM((1568,), jnp.float32),
          pltpu.SemaphoreType.DMA,
          pltpu.SemaphoreType.DMA,
          pltpu.SemaphoreType.DMA,
          pltpu.SemaphoreType.DMA,
      ],
      compiler_params=pltpu.CompilerParams(needs_layout_passes=False,
                                           use_tc_tiling_on_sc=False),
  )
  return kfn(row, col, hp, degp_flat)


GRU_BLK = 512
GRU_NB = FLAT3 // GRU_BLK


def _gru_body(x_ref, wih_ref, whh_ref, bih_ref, bhh_ref, wout_ref, gi_s, gh_s):
  i = pl.program_id(0)
  xv = x_ref[...]
  dn = (((1,), (1,)), ((), ()))
  gi = lax.dot_general(xv, wih_ref[...], dn,
                       preferred_element_type=jnp.float32) + bih_ref[...]
  gh = lax.dot_general(xv, whh_ref[...], dn,
                       preferred_element_type=jnp.float32) + bhh_ref[...]
  gi_s[:, pl.ds(i * GRU_BLK, GRU_BLK)] = gi
  gh_s[:, pl.ds(i * GRU_BLK, GRU_BLK)] = gh

  @pl.when(i == GRU_NB - 1)
  def _():
    r = jax.nn.sigmoid(gi_s[:, 0:FLAT] + gh_s[:, 0:FLAT])
    z = jax.nn.sigmoid(gi_s[:, FLAT:2 * FLAT] + gh_s[:, FLAT:2 * FLAT])
    n = jnp.tanh(gi_s[:, 2 * FLAT:] + r * gh_s[:, 2 * FLAT:])
    wout_ref[...] = (1.0 - z) * n + z * xv


def _gru_call(x_flat, W_ih, W_hh, b_ih, b_hh):
  return pl.pallas_call(
      _gru_body,
      grid=(GRU_NB,),
      in_specs=[
          pl.BlockSpec((1, FLAT), lambda i: (0, 0)),
          pl.BlockSpec((GRU_BLK, FLAT), lambda i: (i, 0)),
          pl.BlockSpec((GRU_BLK, FLAT), lambda i: (i, 0)),
          pl.BlockSpec((1, GRU_BLK), lambda i: (0, i)),
          pl.BlockSpec((1, GRU_BLK), lambda i: (0, i)),
      ],
      out_specs=pl.BlockSpec((1, FLAT), lambda i: (0, 0)),
      out_shape=jax.ShapeDtypeStruct((1, FLAT), jnp.float32),
      scratch_shapes=[
          pltpu.VMEM((1, FLAT3), jnp.float32),
          pltpu.VMEM((1, FLAT3), jnp.float32),
      ],
      compiler_params=pltpu.CompilerParams(
          vmem_limit_bytes=48 * 1024 * 1024),
  )(x_flat, W_ih, W_hh, b_ih, b_hh)


ROW_BLK = 2048
ROW_NB = (N + ROW_BLK - 1) // ROW_BLK


def _hp_body(xt_ref, w_ref, degp_ref, hp_ref):
  d = lax.rsqrt(degp_ref[:, 0] + degp_ref[:, 1] + 1.0)
  h = lax.dot_general(xt_ref[...], w_ref[...], (((0,), (0,)), ((), ())),
                      preferred_element_type=jnp.float32)
  hp_ref[...] = h * d[:, None]


def _hp_call(xt, weight, degp):
  return pl.pallas_call(
      _hp_body,
      grid=(ROW_NB,),
      in_specs=[
          pl.BlockSpec((IN_DIM, ROW_BLK), lambda i: (0, i)),
          pl.BlockSpec((IN_DIM, OUT_DIM), lambda i: (0, 0)),
          pl.BlockSpec((ROW_BLK, NC), lambda i: (i, 0)),
      ],
      out_specs=pl.BlockSpec((ROW_BLK, OUT_DIM), lambda i: (i, 0)),
      out_shape=jax.ShapeDtypeStruct((N, OUT_DIM), jnp.float32),
  )(xt, weight, degp)


def _final_body(acc_ref, degp_ref, out_ref):
  d = lax.rsqrt(degp_ref[:, 0] + degp_ref[:, 1] + 1.0)
  out_ref[...] = jnp.maximum(acc_ref[...] * d[:, None], 0.0)


def _final_call(accum, degp):
  return pl.pallas_call(
      _final_body,
      grid=(ROW_NB,),
      in_specs=[
          pl.BlockSpec((ROW_BLK, OUT_DIM), lambda i: (i, 0)),
          pl.BlockSpec((ROW_BLK, NC), lambda i: (i, 0)),
      ],
      out_specs=pl.BlockSpec((ROW_BLK, OUT_DIM), lambda i: (i, 0)),
      out_shape=jax.ShapeDtypeStruct((N, OUT_DIM), jnp.float32),
  )(accum, degp)


def kernel(x, edge_index, initial_weight, W_ih, W_hh, b_ih, b_hh):
  x_flat = initial_weight.reshape(1, FLAT)
  row, col = edge_index[0], edge_index[1]
  degp_flat = _deg_call(row, col)
  degp = degp_flat.reshape(NC, N).T
  wnew = _gru_call(x_flat, W_ih, W_hh, b_ih.reshape(1, FLAT3),
                   b_hh.reshape(1, FLAT3))
  hp = _hp_call(x.T, wnew.reshape(IN_DIM, OUT_DIM), degp)
  return _scatter_call(row, col, hp, degp_flat)


# speedup vs baseline: 34.1414x; 1.0005x over previous
import jax
import jax.numpy as jnp
from jax import lax
from jax.experimental import pallas as pl
from jax.experimental.pallas import tpu as pltpu
from jax.experimental.pallas import tpu_sc as plsc

N = 50000
E = 800000
IN_DIM = 64
OUT_DIM = 64
FLAT = IN_DIM * OUT_DIM
FLAT3 = 3 * FLAT

NC = 2
NS = 16
L = 16

HALF = N // NC
DUMMY = HALF
ACC_ROWS = HALF + 8

EDGES_PER_TILE = E // (NC * NS)
A_CH = 2000
A_NFULL = 12
A_TAIL = EDGES_PER_TILE - A_NFULL * A_CH
A_PAD = 25088
DEG_SH = 50176
DEG_SLICE = DEG_SH // NS

EDGES_PER_SUB = E // NS
C_CH = 2000
C_NCH = EDGES_PER_SUB // C_CH
C_BUF = 2304
C_TRASH = 2288
G = 128


def _deg_body(row_ref, col_ref, degp_ref, deg_sh, zbuf, rstage, cstage, valbuf,
              colbuf2, sem_s):
  c = lax.axis_index("c")
  s = lax.axis_index("s")
  tile = c * NS + s
  base_e = tile * EDGES_PER_TILE

  zeros16 = jnp.zeros((L,), jnp.float32)
  for g in range(DEG_SLICE // L):
    zbuf[pl.ds(g * L, L)] = zeros16
  pltpu.sync_copy(zbuf, deg_sh.at[pl.ds(s * DEG_SLICE, DEG_SLICE)])
  plsc.subcore_barrier()

  lane = lax.iota(jnp.int32, L)

  def scan_chunk(off_e, off_buf, n_groups):
    pltpu.sync_copy(row_ref.at[pl.ds(off_e, A_CH)],
                    rstage.at[pl.ds(0, A_CH)])
    pltpu.sync_copy(col_ref.at[pl.ds(off_e, A_CH)],
                    cstage.at[pl.ds(0, A_CH)])

    def grp(j, _):
      r = rstage[pl.ds(j * L, L)]
      cv = cstage[pl.ds(j * L, L)]
      val = jnp.where(r != cv, 1.0, 0.0).astype(jnp.float32)
      off = off_buf + j * L
      valbuf[pl.ds(off, L)] = val
      colbuf2[lax.shift_right_logical(off, 7), pl.ds(lax.bitwise_and(off, G - 1), L)] = cv
      return 0

    lax.fori_loop(0, n_groups, grp, 0)

  def chunk_loop(k, _):
    scan_chunk(base_e + k * A_CH, k * A_CH, A_CH // L)
    return 0

  lax.fori_loop(0, A_NFULL, chunk_loop, 0)

  toff = base_e + A_NFULL * A_CH
  tbuf = A_NFULL * A_CH
  pltpu.sync_copy(row_ref.at[pl.ds(toff, A_TAIL)],
                  rstage.at[pl.ds(0, A_TAIL)])
  pltpu.sync_copy(col_ref.at[pl.ds(toff, A_TAIL)],
                  cstage.at[pl.ds(0, A_TAIL)])

  def tgrp(j, _):
    r = rstage[pl.ds(j * L, L)]
    cv = cstage[pl.ds(j * L, L)]
    val = jnp.where(r != cv, 1.0, 0.0).astype(jnp.float32)
    off = tbuf + j * L
    valbuf[pl.ds(off, L)] = val
    colbuf2[off // G, pl.ds(off % G, L)] = cv
    return 0

  lax.fori_loop(0, (A_TAIL - 8) // L, tgrp, 0)

  jt = (A_TAIL - 8) // L
  r = rstage[pl.ds(jt * L, L)]
  cv = cstage[pl.ds(jt * L, L)]
  ok = lane < 8
  val = jnp.where((r != cv) & ok, 1.0, 0.0).astype(jnp.float32)
  cv = jnp.where(ok, cv, 0)
  offt = tbuf + jt * L
  valbuf[pl.ds(offt, L)] = val
  colbuf2[lax.shift_right_logical(offt, 7), pl.ds(lax.bitwise_and(offt, G - 1), L)] = cv

  zi16 = jnp.zeros((L,), jnp.int32)
  for t in range((A_PAD - (EDGES_PER_TILE + 8)) // L):
    flat = EDGES_PER_TILE + 8 + t * L
    valbuf[pl.ds(flat, L)] = zeros16
    colbuf2[flat // G, pl.ds(flat % G, L)] = zi16

  def scat(g, _):
    pltpu.async_copy(valbuf.at[pl.ds(g * G, G)], deg_sh.at[colbuf2.at[g]],
                     sem_s, add=True)
    return 0

  lax.fori_loop(0, A_PAD // G, scat, 0)

  def drain(g, _):
    pltpu.make_async_copy(valbuf.at[pl.ds(0, G)], deg_sh.at[colbuf2.at[0]],
                          sem_s).wait()
    return 0

  lax.fori_loop(0, A_PAD // G, drain, 0)
  plsc.subcore_barrier()

  pltpu.sync_copy(deg_sh.at[pl.ds(s * DEG_SLICE, DEG_SLICE)], zbuf)

  @pl.when(s < NS - 1)
  def _():
    pltpu.sync_copy(zbuf, degp_ref.at[pl.ds(c * N + s * DEG_SLICE, DEG_SLICE)])

  @pl.when(s == NS - 1)
  def _():
    last = N - (NS - 1) * DEG_SLICE
    pltpu.sync_copy(zbuf.at[pl.ds(0, last)],
                    degp_ref.at[pl.ds(c * N + (NS - 1) * DEG_SLICE, last)])


def _deg_call(row, col):
  kfn = pl.kernel(
      _deg_body,
      out_type=jax.ShapeDtypeStruct((NC * N,), jnp.float32),
      mesh=plsc.VectorSubcoreMesh(core_axis_name="c", subcore_axis_name="s"),
      scratch_types=[
          pltpu.VMEM_SHARED((DEG_SH,), jnp.float32),
          pltpu.VMEM((DEG_SLICE,), jnp.float32),
          pltpu.VMEM((A_CH + 16,), jnp.int32),
          pltpu.VMEM((A_CH + 16,), jnp.int32),
          pltpu.VMEM((A_PAD,), jnp.float32),
          pltpu.VMEM((A_PAD // G, G), jnp.int32),
          pltpu.SemaphoreType.DMA,
      ],
  )
  return kfn(row, col)


def _newton_rsqrt(x):
  i = plsc.bitcast(x, jnp.int32)
  y = plsc.bitcast(jnp.int32(0x5F3759DF) - lax.shift_right_logical(i, 1),
                   jnp.float32)
  y = y * (1.5 - 0.5 * x * y * y)
  y = y * (1.5 - 0.5 * x * y * y)
  y = y * (1.5 - 0.5 * x * y * y)
  return y


def _scatter_body(row_ref, col_ref, hp_ref, degp_ref, out_ref, accum, rstage,
                  cstage, rbuf, cbuf2, grows, grows_b, dbuf, dstage0, dstage1,
                  sem_r, sem_c, sem_ga, sem_gb):
  c = lax.axis_index("c")
  s = lax.axis_index("s")
  base = c * HALF

  def init_chunk(r0, nrows):
    pltpu.sync_copy(hp_ref.at[pl.ds(base + r0, nrows)],
                    grows.at[pl.ds(0, nrows)])
    pltpu.sync_copy(grows.at[pl.ds(0, nrows)], accum.at[pl.ds(r0, nrows)])

  @pl.when(s < NS - 1)
  def _():
    def body(q, _):
      init_chunk(s * 1568 + q * G, G)
      return 0
    lax.fori_loop(0, 12, body, 0)
    init_chunk(s * 1568 + 12 * G, 32)

  @pl.when(s == NS - 1)
  def _():
    def body(q, _):
      init_chunk(15 * 1568 + q * G, G)
      return 0
    lax.fori_loop(0, 11, body, 0)
    init_chunk(15 * 1568 + 11 * G, 72)

  plsc.subcore_barrier()

  zeros16 = jnp.zeros((L,), jnp.int32)
  dummy16 = jnp.full((L,), DUMMY, jnp.int32)
  lane = lax.iota(jnp.int32, L)

  def wait_gather(buf, sem):
    pltpu.make_async_copy(hp_ref.at[rbuf.at[pl.ds(0, G)]], buf, sem).wait()

  def chunk(k, p_in):
    off = s * EDGES_PER_SUB + k * C_CH
    d1 = pltpu.async_copy(row_ref.at[pl.ds(off, C_CH)], rstage, sem_r)
    d2 = pltpu.async_copy(col_ref.at[pl.ds(off, C_CH)], cstage, sem_c)
    d1.wait()
    d2.wait()

    def grp(j, p):
      r = rstage[pl.ds(j * L, L)]
      cv = cstage[pl.ds(j * L, L)]
      cl = cv - base
      m = (cl >= 0) & (cl < HALF) & (r != cv)
      cum = plsc.cumsum(m.astype(jnp.int32))
      pos = jnp.where(m, p + cum - 1, C_TRASH + lane)
      plsc.store_scatter(rbuf, [pos], r)
      plsc.store_scatter(cbuf2, [lax.shift_right_logical(pos, 7),
                                  lax.bitwise_and(pos, G - 1)], cl)
      return p + jnp.max(cum)

    cnt = lax.fori_loop(0, C_CH // L, grp, p_in)
    ngf = cnt // G

    @pl.when(ngf > 0)
    def _():
      pltpu.async_copy(hp_ref.at[rbuf.at[pl.ds(0, G)]], grows, sem_ga)

      def do_group(g, _):
        even = (g % 2) == 0

        @pl.when(g + 1 < ngf)
        def _():
          @pl.when(even)
          def _():
            pltpu.async_copy(hp_ref.at[rbuf.at[pl.ds((g + 1) * G, G)]],
                             grows_b, sem_gb)

          @pl.when(jnp.logical_not(even))
          def _():
            pltpu.async_copy(hp_ref.at[rbuf.at[pl.ds((g + 1) * G, G)]],
                             grows, sem_ga)

        @pl.when(even)
        def _():
          wait_gather(grows, sem_ga)
          pltpu.sync_copy(grows, accum.at[cbuf2.at[g]], add=True)

        @pl.when(jnp.logical_not(even))
        def _():
          wait_gather(grows_b, sem_gb)
          pltpu.sync_copy(grows_b, accum.at[cbuf2.at[g]], add=True)

        return 0

      lax.fori_loop(0, ngf, do_group, 0)

      for t in range(G // L):
        tmp = rbuf[pl.ds(ngf * G + t * L, L)]
        rbuf[pl.ds(t * L, L)] = tmp
        tmp2 = cbuf2[ngf, pl.ds(t * L, L)]
        cbuf2[0, pl.ds(t * L, L)] = tmp2

    return cnt - ngf * G

  rem = lax.fori_loop(0, C_NCH, chunk, jnp.int32(0))


  @pl.when(s < NS - 1)
  def _():
    pltpu.sync_copy(degp_ref.at[pl.ds(base + s * 1568, 1568)], dstage0)
    pltpu.sync_copy(degp_ref.at[pl.ds(N + base + s * 1568, 1568)], dstage1)

  @pl.when(s == NS - 1)
  def _():
    pltpu.sync_copy(degp_ref.at[pl.ds(base + 15 * 1568, 1480)],
                    dstage0.at[pl.ds(0, 1480)])
    pltpu.sync_copy(degp_ref.at[pl.ds(N + base + 15 * 1568, 1480)],
                    dstage1.at[pl.ds(0, 1480)])

  @pl.when(rem > 0)
  def _():
    for t in range(G // L):
      padpos = rem + t * L + lane
      rbuf[pl.ds(rem + t * L, L)] = zeros16
      plsc.store_scatter(cbuf2, [lax.shift_right_logical(padpos, 7),
                                  lax.bitwise_and(padpos, G - 1)], dummy16)
    pltpu.sync_copy(hp_ref.at[rbuf.at[pl.ds(0, G)]], grows)
    pltpu.sync_copy(grows, accum.at[cbuf2.at[0]], add=True)

  plsc.subcore_barrier()

  def out_chunk(r0, lo, nrows):
    pltpu.sync_copy(accum.at[pl.ds(r0, nrows)], grows.at[pl.ds(0, nrows)])

    def dgrp(jj, _):
      d0 = dstage0[pl.ds(lo + jj * L, L)]
      d1 = dstage1[pl.ds(lo + jj * L, L)]
      dbuf[pl.ds(jj * L, L)] = _newton_rsqrt(d0 + d1 + 1.0)
      return 0

    lax.fori_loop(0, (nrows + L - 1) // L, dgrp, 0)

    def rowf(rr, _):
      g16 = lax.bitwise_and(rr, jnp.int32(~(L - 1)))
      dv = dbuf[pl.ds(g16, L)]
      dsp = jnp.max(jnp.where(lane == lax.bitwise_and(rr, jnp.int32(L - 1)),
                              dv, jnp.float32(-3e38)))
      for seg in range(OUT_DIM // L):
        v = grows[rr, pl.ds(seg * L, L)]
        grows[rr, pl.ds(seg * L, L)] = jnp.maximum(v * dsp, 0.0)
      return 0

    lax.fori_loop(0, nrows, rowf, 0)
    pltpu.sync_copy(grows.at[pl.ds(0, nrows)],
                    out_ref.at[pl.ds(base + r0, nrows)])

  @pl.when(s < NS - 1)
  def _():
    def body(q, _):
      out_chunk(s * 1568 + q * G, q * G, G)
      return 0
    lax.fori_loop(0, 12, body, 0)
    out_chunk(s * 1568 + 12 * G, 12 * G, 32)

  @pl.when(s == NS - 1)
  def _():
    def body(q, _):
      out_chunk(15 * 1568 + q * G, q * G, G)
      return 0
    lax.fori_loop(0, 11, body, 0)
    out_chunk(15 * 1568 + 11 * G, 11 * G, 72)


def _scatter_call(row, col, hp, degp_flat):
  kfn = pl.kernel(
      _scatter_body,
      out_type=jax.ShapeDtypeStruct((N, OUT_DIM), jnp.float32),
      mesh=plsc.VectorSubcoreMesh(core_axis_name="c", subcore_axis_name="s"),
      scratch_types=[
          pltpu.VMEM_SHARED((ACC_ROWS, OUT_DIM), jnp.float32),
          pltpu.VMEM((C_CH,), jnp.int32),
          pltpu.VMEM((C_CH,), jnp.int32),
          pltpu.VMEM((C_BUF,), jnp.int32),
          pltpu.VMEM((18, G), jnp.int32),
          pltpu.VMEM((G, OUT_DIM), jnp.float32),
          pltpu.VMEM((G, OUT_DIM), jnp.float32),
          pltpu.VMEM((G,), jnp.float32),
          pltpu.VMEM((1568,), jnp.float32),
          pltpu.VMEM((1568,), jnp.float32),
          pltpu.SemaphoreType.DMA,
          pltpu.SemaphoreType.DMA,
          pltpu.SemaphoreType.DMA,
          pltpu.SemaphoreType.DMA,
      ],
      compiler_params=pltpu.CompilerParams(needs_layout_passes=False,
                                           use_tc_tiling_on_sc=False),
  )
  return kfn(row, col, hp, degp_flat)


GRU_BLK = 512
GRU_NB = FLAT3 // GRU_BLK


def _gru_body(x_ref, wih_ref, whh_ref, bih_ref, bhh_ref, wout_ref, gi_s, gh_s):
  i = pl.program_id(0)
  xv = x_ref[...]
  dn = (((1,), (1,)), ((), ()))
  gi = lax.dot_general(xv, wih_ref[...], dn,
                       preferred_element_type=jnp.float32) + bih_ref[...]
  gh = lax.dot_general(xv, whh_ref[...], dn,
                       preferred_element_type=jnp.float32) + bhh_ref[...]
  gi_s[:, pl.ds(i * GRU_BLK, GRU_BLK)] = gi
  gh_s[:, pl.ds(i * GRU_BLK, GRU_BLK)] = gh

  @pl.when(i == GRU_NB - 1)
  def _():
    r = jax.nn.sigmoid(gi_s[:, 0:FLAT] + gh_s[:, 0:FLAT])
    z = jax.nn.sigmoid(gi_s[:, FLAT:2 * FLAT] + gh_s[:, FLAT:2 * FLAT])
    n = jnp.tanh(gi_s[:, 2 * FLAT:] + r * gh_s[:, 2 * FLAT:])
    wout_ref[...] = (1.0 - z) * n + z * xv


def _gru_call(x_flat, W_ih, W_hh, b_ih, b_hh):
  return pl.pallas_call(
      _gru_body,
      grid=(GRU_NB,),
      in_specs=[
          pl.BlockSpec((1, FLAT), lambda i: (0, 0)),
          pl.BlockSpec((GRU_BLK, FLAT), lambda i: (i, 0)),
          pl.BlockSpec((GRU_BLK, FLAT), lambda i: (i, 0)),
          pl.BlockSpec((1, GRU_BLK), lambda i: (0, i)),
          pl.BlockSpec((1, GRU_BLK), lambda i: (0, i)),
      ],
      out_specs=pl.BlockSpec((1, FLAT), lambda i: (0, 0)),
      out_shape=jax.ShapeDtypeStruct((1, FLAT), jnp.float32),
      scratch_shapes=[
          pltpu.VMEM((1, FLAT3), jnp.float32),
          pltpu.VMEM((1, FLAT3), jnp.float32),
      ],
      compiler_params=pltpu.CompilerParams(
          vmem_limit_bytes=48 * 1024 * 1024),
  )(x_flat, W_ih, W_hh, b_ih, b_hh)


ROW_BLK = 2048
ROW_NB = (N + ROW_BLK - 1) // ROW_BLK


def _hp_body(xt_ref, w_ref, degp_ref, hp_ref):
  d = lax.rsqrt(degp_ref[:, 0] + degp_ref[:, 1] + 1.0)
  h = lax.dot_general(xt_ref[...], w_ref[...], (((0,), (0,)), ((), ())),
                      preferred_element_type=jnp.float32)
  hp_ref[...] = h * d[:, None]


def _hp_call(xt, weight, degp):
  return pl.pallas_call(
      _hp_body,
      grid=(ROW_NB,),
      in_specs=[
          pl.BlockSpec((IN_DIM, ROW_BLK), lambda i: (0, i)),
          pl.BlockSpec((IN_DIM, OUT_DIM), lambda i: (0, 0)),
          pl.BlockSpec((ROW_BLK, NC), lambda i: (i, 0)),
      ],
      out_specs=pl.BlockSpec((ROW_BLK, OUT_DIM), lambda i: (i, 0)),
      out_shape=jax.ShapeDtypeStruct((N, OUT_DIM), jnp.float32),
  )(xt, weight, degp)


def kernel(x, edge_index, initial_weight, W_ih, W_hh, b_ih, b_hh):
  x_flat = initial_weight.reshape(1, FLAT)
  row, col = edge_index[0], edge_index[1]
  degp_flat = _deg_call(row, col)
  degp = degp_flat.reshape(NC, N).T
  wnew = _gru_call(x_flat, W_ih, W_hh, b_ih.reshape(1, FLAT3),
                   b_hh.reshape(1, FLAT3))
  hp = _hp_call(x.T, wnew.reshape(IN_DIM, OUT_DIM), degp)
  return _scatter_call(row, col, hp, degp_flat)
